# Initial kernel scaffold; baseline (speedup 1.0000x reference)
import jax
import jax.numpy as jnp
from jax.experimental import pallas as pl


def kernel(x, edge_index, edge_weight, W):
    raise NotImplementedError("write your pallas kernel here")


# speedup vs baseline: 2.3161x; 2.3161x over previous
import functools

import jax
import jax.numpy as jnp
from jax import lax
from jax.experimental import pallas as pl
from jax.experimental.pallas import tpu as pltpu
from jax.experimental.pallas import tpu_sc as plsc

NC = 2
NS = 16
L = 16

EDGE_BLK = 80
ROW_BLK = 80


@functools.partial(jax.jit, static_argnames=("n", "e", "d", "h"))
def _diffusion_sc(x, src, dst, ew, w, *, n, e, d, h):
    f = d // NC
    epw1 = e // NS
    n_chunks1 = epw1 // EDGE_BLK
    epw2 = e // (NC * NS)
    n_chunks2 = epw2 // EDGE_BLK
    n_row_chunks = -(-n // ROW_BLK)
    row_chunks_per_tile = -(-n_row_chunks // NS)

    mesh = plsc.VectorSubcoreMesh(
        core_axis_name="c", subcore_axis_name="s", num_cores=NC,
        num_subcores=NS)

    @functools.partial(
        pl.kernel,
        out_type=(
            jax.ShapeDtypeStruct((NC, 2, n, f), jnp.float32),
            jax.ShapeDtypeStruct((NC, n, d), jnp.float32),
        ),
        mesh=mesh,
        scratch_types=[
            pltpu.VMEM((EDGE_BLK,), jnp.int32),
            pltpu.VMEM((EDGE_BLK,), jnp.int32),
            pltpu.VMEM((EDGE_BLK * 2,), jnp.float32),
            pltpu.VMEM((EDGE_BLK, d), jnp.float32),
            pltpu.VMEM((EDGE_BLK, d), jnp.float32),
            pltpu.VMEM((2, ROW_BLK, f), jnp.float32),
            pltpu.VMEM((2, f), jnp.float32),
            pltpu.VMEM_SHARED((n, d), jnp.float32),
            pltpu.SemaphoreType.DMA,
        ],
    )
    def k(x_hbm, src_hbm, dst_hbm, ew_hbm, w_hbm,
          out01_hbm, part_hbm,
          sidx, didx, wtv, rows, msg, obuf, wbuf, acc, sem):
        c = lax.axis_index("c")
        s = lax.axis_index("s")
        zeros = jnp.zeros((L,), jnp.float32)
        lanes = [jnp.full((L,), ll, jnp.int32) for ll in range(L)]

        def bcast(vec, ll):
            return jnp.take_along_axis(vec, lanes[ll], axis=0)

        def zero_rows(r, _):
            for j in range(d // L):
                rows[r, pl.ds(j * L, L)] = zeros
            return 0

        def zero_acc():
            for kk in range(row_chunks_per_tile):
                cid = s + NS * kk
                @pl.when(cid < n_row_chunks)
                def _():
                    pltpu.sync_copy(rows, acc.at[pl.ds(cid * ROW_BLK, ROW_BLK)])

        lax.fori_loop(0, ROW_BLK, zero_rows, 0)
        zero_acc()
        plsc.subcore_barrier()

        def edge_chunk1(kk, _):
            off = pl.multiple_of(s * epw1 + kk * EDGE_BLK, EDGE_BLK)
            pltpu.sync_copy(src_hbm.at[pl.ds(off, EDGE_BLK)], sidx)
            pltpu.sync_copy(dst_hbm.at[pl.ds(off, EDGE_BLK)], didx)
            pltpu.sync_copy(ew_hbm.at[pl.ds(off, EDGE_BLK)],
                            wtv.at[pl.ds(0, EDGE_BLK)])
            pltpu.sync_copy(ew_hbm.at[pl.ds(e + off, EDGE_BLK)],
                            wtv.at[pl.ds(EDGE_BLK, EDGE_BLK)])
            pltpu.async_copy(x_hbm.at[sidx], rows, sem).wait()

            def one_group(g, _):
                w0v = wtv[pl.ds(g * L, L)]
                w1v = wtv[pl.ds(EDGE_BLK + g * L, L)]
                for ll in range(L):
                    ei = g * L + ll
                    r = [rows[ei, pl.ds(c * f + j * L, L)]
                         for j in range(f // L)]
                    w0 = bcast(w0v, ll)
                    w1 = bcast(w1v, ll)
                    for j in range(f // L):
                        msg[ei, pl.ds(j * L, L)] = w0 * r[j]
                        msg[ei, pl.ds(f + j * L, L)] = w1 * r[j]
                return 0
            lax.fori_loop(0, EDGE_BLK // L, one_group, 0)
            pltpu.sync_copy(msg, acc.at[didx], add=True)
            return 0
        lax.fori_loop(0, n_chunks1, edge_chunk1, 0)
        plsc.subcore_barrier()

        for hh in range(2):
            pltpu.sync_copy(w_hbm.at[pl.ds(hh * d + c * f, f)], wbuf.at[hh])
        wv = [[wbuf[hh, pl.ds(j * L, L)] for j in range(f // L)]
              for hh in range(2)]
        for kk in range(row_chunks_per_tile):
            cid = s + NS * kk
            @pl.when(cid < n_row_chunks)
            def _():
                r0 = pl.multiple_of(cid * ROW_BLK, ROW_BLK)
                pltpu.sync_copy(acc.at[pl.ds(r0, ROW_BLK)], rows)

                def one_row(ri, _):
                    for hh in range(2):
                        for j in range(f // L):
                            v = rows[ri, pl.ds(hh * f + j * L, L)]
                            obuf[hh, ri, pl.ds(j * L, L)] = jnp.maximum(
                                wv[hh][j] * v, 0.0)
                    return 0
                lax.fori_loop(0, ROW_BLK, one_row, 0)
                for hh in range(2):
                    pltpu.sync_copy(obuf.at[hh],
                                    out01_hbm.at[c, hh, pl.ds(r0, ROW_BLK)])
        plsc.subcore_barrier()

        lax.fori_loop(0, ROW_BLK, zero_rows, 0)
        zero_acc()
        plsc.subcore_barrier()

        def edge_chunk2(kk, _):
            off = pl.multiple_of(
                c * (NS * epw2) + s * epw2 + kk * EDGE_BLK, EDGE_BLK)
            pltpu.sync_copy(src_hbm.at[pl.ds(off, EDGE_BLK)], sidx)
            pltpu.sync_copy(dst_hbm.at[pl.ds(off, EDGE_BLK)], didx)
            pltpu.sync_copy(ew_hbm.at[pl.ds(2 * e + off, EDGE_BLK)],
                            wtv.at[pl.ds(0, EDGE_BLK)])
            pltpu.async_copy(x_hbm.at[sidx], rows, sem).wait()

            def one_group(g, _):
                w2v = wtv[pl.ds(g * L, L)]
                for ll in range(L):
                    ei = g * L + ll
                    w2 = bcast(w2v, ll)
                    for j in range(d // L):
                        msg[ei, pl.ds(j * L, L)] = (
                            w2 * rows[ei, pl.ds(j * L, L)])
                return 0
            lax.fori_loop(0, EDGE_BLK // L, one_group, 0)
            pltpu.sync_copy(msg, acc.at[didx], add=True)
            return 0
        lax.fori_loop(0, n_chunks2, edge_chunk2, 0)
        plsc.subcore_barrier()

        for kk in range(row_chunks_per_tile):
            cid = s + NS * kk
            @pl.when(cid < n_row_chunks)
            def _():
                r0 = pl.multiple_of(cid * ROW_BLK, ROW_BLK)
                pltpu.sync_copy(acc.at[pl.ds(r0, ROW_BLK)],
                                part_hbm.at[c, pl.ds(r0, ROW_BLK)])

    return k(x, src, dst, ew, w)


def kernel(x, edge_index, edge_weight, W):
    n, d = x.shape
    h, e = edge_weight.shape
    src = edge_index[1].astype(jnp.int32)
    dst = edge_index[0].astype(jnp.int32)
    ew = edge_weight.reshape(-1)
    out01, part = _diffusion_sc(
        x, src, dst, ew, W.reshape(-1), n=n, e=e, d=d, h=h)
    first = out01.transpose(1, 2, 0, 3).reshape(2, n, d)
    last = jax.nn.relu(W[2] * (part[0] + part[1]))
    return jnp.concatenate([first, last[None]], axis=0)
